# baseline (speedup 1.0000x reference)
import functools

import jax
import jax.numpy as jnp
from jax import lax
from jax.experimental import pallas as pl
from jax.experimental.pallas import tpu as pltpu
from jax.experimental.pallas import tpu_sc as plsc

B, H, N = 32, 8, 100000
ROWS = B * H
L = 16
NC, NS = 2, 16
NW = NC * NS
RPW = ROWS // NW
SPAN = RPW * N
CH = 32000
NCH = SPAN // CH
CHV = CH // L


def _chunk_segments(ci):
    lo, hi = ci * CH, (ci + 1) * CH
    segs = []
    pos = lo
    while pos < hi:
        row = pos // N
        seg_end = min(hi, (row + 1) * N)
        segs.append((
            (pos - lo) // L,
            (seg_end - lo) // L,
            lo - row * N,
            row if seg_end == (row + 1) * N else None,
        ))
        pos = seg_end
    return segs


@functools.partial(
    pl.kernel,
    mesh=plsc.VectorSubcoreMesh(core_axis_name="c", subcore_axis_name="s"),
    out_type=jax.ShapeDtypeStruct((NW * L,), jnp.float32),
    scratch_types=[
        pltpu.VMEM((2, CH), jnp.float32),
        pltpu.VMEM((L,), jnp.float32),
        pltpu.SemaphoreType.DMA,
        pltpu.SemaphoreType.DMA,
    ],
)
def _argmax_sc(x_hbm, out_hbm, buf, res, sem0, sem1):
    c = lax.axis_index("c")
    s = lax.axis_index("s")
    wid = s * NC + c
    base = wid * SPAN
    sems = (sem0, sem1)
    iota = lax.iota(jnp.int32, L)

    def hslice(ci):
        off = pl.multiple_of(base + ci * CH, 128)
        return x_hbm.at[pl.ds(off, CH)]

    cps = [None, None]
    cps[0] = pltpu.async_copy(hslice(0), buf.at[0], sems[0])

    m = jnp.full((L,), -jnp.inf, dtype=jnp.float32)
    am = jnp.zeros((L,), dtype=jnp.int32)
    resv = jnp.zeros((L,), dtype=jnp.float32)

    for ci in range(NCH):
        b = ci % 2
        if ci + 1 < NCH:
            nb = (ci + 1) % 2
            cps[nb] = pltpu.async_copy(hslice(ci + 1), buf.at[nb], sems[nb])
        cps[b].wait()

        for sv, ev, ibase, row_end in _chunk_segments(ci):

            def body(i, carry, b=b, ibase=ibase):
                mm, aa = carry
                v = buf[b, pl.ds(i * L, L)]
                idxv = iota + (ibase + i * L)
                gt = v > mm
                mm = jnp.where(gt, v, mm)
                aa = jnp.where(gt, idxv, aa)
                return mm, aa

            m, am = lax.fori_loop(sv, ev, body, (m, am))

            if row_end is not None:
                mm, aa = m, am
                for sh in (8, 4, 2, 1):
                    perm = iota ^ sh
                    mo = mm.at[perm].get(mode="promise_in_bounds")
                    ao = aa.at[perm].get(mode="promise_in_bounds")
                    better = (mo > mm) | ((mo == mm) & (ao < aa))
                    mm = jnp.where(better, mo, mm)
                    aa = jnp.where(better, ao, aa)
                val = aa.astype(jnp.float32) / jnp.float32(N)
                resv = jnp.where(iota == (row_end % RPW), val, resv)
                m = jnp.full((L,), -jnp.inf, dtype=jnp.float32)
                am = jnp.zeros((L,), dtype=jnp.int32)

    res[...] = resv
    oout = pl.multiple_of(wid * L, 8)
    pltpu.sync_copy(res, out_hbm.at[pl.ds(oout, L)])


def kernel(x):
    x1 = x.reshape(ROWS * N)
    out = _argmax_sc(x1)
    return out.reshape(NW, L)[:, :RPW].reshape(B, H)


# speedup vs baseline: 1.5473x; 1.5473x over previous
import functools

import jax
import jax.numpy as jnp
from jax import lax
from jax.experimental import pallas as pl
from jax.experimental.pallas import tpu as pltpu
from jax.experimental.pallas import tpu_sc as plsc

B, H, N = 32, 8, 100000
ROWS = B * H
L = 16
NC, NS = 2, 16
NW = NC * NS
RPW = ROWS // NW
SPAN = RPW * N
CH = 32000
NCH = SPAN // CH
CHV = CH // L
U = 10


def _chunk_segments(ci):
    lo, hi = ci * CH, (ci + 1) * CH
    segs = []
    pos = lo
    while pos < hi:
        row = pos // N
        seg_end = min(hi, (row + 1) * N)
        segs.append((
            (pos - lo) // L,
            (seg_end - lo) // L,
            lo - row * N,
            row if seg_end == (row + 1) * N else None,
        ))
        pos = seg_end
    return segs


@functools.partial(
    pl.kernel,
    mesh=plsc.VectorSubcoreMesh(core_axis_name="c", subcore_axis_name="s"),
    out_type=jax.ShapeDtypeStruct((NW * L,), jnp.float32),
    scratch_types=[
        pltpu.VMEM((2, CH), jnp.float32),
        pltpu.VMEM((L,), jnp.float32),
        pltpu.SemaphoreType.DMA,
        pltpu.SemaphoreType.DMA,
    ],
)
def _argmax_sc(x_hbm, out_hbm, buf, res, sem0, sem1):
    c = lax.axis_index("c")
    s = lax.axis_index("s")
    wid = s * NC + c
    base = wid * SPAN
    sems = (sem0, sem1)
    iota = lax.iota(jnp.int32, L)
    neg_inf = jnp.full((L,), -jnp.inf, dtype=jnp.float32)
    zeros_i = jnp.zeros((L,), dtype=jnp.int32)
    stepv = jnp.full((L,), U * L, dtype=jnp.int32)

    def hslice(ci):
        off = pl.multiple_of(base + ci * CH, 128)
        return x_hbm.at[pl.ds(off, CH)]

    cps = [None, None]
    cps[0] = pltpu.async_copy(hslice(0), buf.at[0], sems[0])

    ms = [neg_inf] * U
    ams = [zeros_i] * U
    resv = jnp.zeros((L,), dtype=jnp.float32)

    for ci in range(NCH):
        b = ci % 2
        if ci + 1 < NCH:
            nb = (ci + 1) % 2
            cps[nb] = pltpu.async_copy(hslice(ci + 1), buf.at[nb], sems[nb])
        cps[b].wait()

        for sv, ev, ibase, row_end in _chunk_segments(ci):
            idxs = [iota + (ibase + (sv + j) * L) for j in range(U)]

            def body(i, carry, b=b):
                mm = list(carry[0:U])
                aa = list(carry[U:2 * U])
                ix = list(carry[2 * U:3 * U])
                off = i * (U * L)
                out_ix = []
                for j in range(U):
                    v = buf[b, pl.ds(off + j * L, L)]
                    gt = v > mm[j]
                    mm[j] = jnp.where(gt, v, mm[j])
                    aa[j] = jnp.where(gt, ix[j], aa[j])
                    out_ix.append(ix[j] + stepv)
                return tuple(mm) + tuple(aa) + tuple(out_ix)

            nseg = (ev - sv) // U
            carry = lax.fori_loop(
                sv // U, sv // U + nseg, body,
                tuple(ms) + tuple(ams) + tuple(idxs))
            ms = list(carry[0:U])
            ams = list(carry[U:2 * U])

            if row_end is not None:
                mm = list(ms)
                aa = list(ams)
                k = U
                while k > 1:
                    half = (k + 1) // 2
                    for j in range(k // 2):
                        a, bj = j, j + half
                        better = (mm[bj] > mm[a]) | (
                            (mm[bj] == mm[a]) & (aa[bj] < aa[a]))
                        mm[a] = jnp.where(better, mm[bj], mm[a])
                        aa[a] = jnp.where(better, aa[bj], aa[a])
                    k = half
                rm, ra = mm[0], aa[0]
                for sh in (8, 4, 2, 1):
                    perm = iota ^ sh
                    mo = rm.at[perm].get(mode="promise_in_bounds")
                    ao = ra.at[perm].get(mode="promise_in_bounds")
                    better = (mo > rm) | ((mo == rm) & (ao < ra))
                    rm = jnp.where(better, mo, rm)
                    ra = jnp.where(better, ao, ra)
                val = ra.astype(jnp.float32) / jnp.float32(N)
                resv = jnp.where(iota == row_end, val, resv)
                ms = [neg_inf] * U
                ams = [zeros_i] * U

    res[...] = resv
    oout = pl.multiple_of(wid * L, 8)
    pltpu.sync_copy(res, out_hbm.at[pl.ds(oout, L)])


def kernel(x):
    x1 = x.reshape(ROWS * N)
    out = _argmax_sc(x1)
    return out.reshape(NW, L)[:, :RPW].reshape(B, H)


# speedup vs baseline: 6.0943x; 3.9386x over previous
import functools

import jax
import jax.numpy as jnp
from jax import lax
from jax.experimental import pallas as pl
from jax.experimental.pallas import tpu as pltpu
from jax.experimental.pallas import tpu_sc as plsc

B, H, N = 32, 8, 100000
L = 16
NC, NS = 2, 16
TB = 128
NT = N // TB
WC = 31 * TB
NCH = NT // 31
REM = (NT - NCH * 31) * TB
TAILC = N - NT * TB
GTAIL = (NT * TB) // L
RING = 4


@functools.partial(
    pl.kernel,
    mesh=plsc.VectorSubcoreMesh(core_axis_name="c", subcore_axis_name="s"),
    out_type=jax.ShapeDtypeStruct((B * L,), jnp.float32),
    scratch_types=[
        pltpu.VMEM((RING, H, WC), jnp.float32),
        pltpu.VMEM((H, TB), jnp.float32),
        pltpu.VMEM((L,), jnp.float32),
        pltpu.SemaphoreType.DMA,
        pltpu.SemaphoreType.DMA,
        pltpu.SemaphoreType.DMA,
        pltpu.SemaphoreType.DMA,
        pltpu.SemaphoreType.DMA,
    ],
)
def _argmax_sc(x_hbm, xt_hbm, out_hbm, buf, tbuf, res,
               sem0, sem1, sem2, sem3, semt):
    c = lax.axis_index("c")
    s = lax.axis_index("s")
    b = s * NC + c
    sems = (sem0, sem1, sem2, sem3)
    iota = lax.iota(jnp.int32, L)

    widths = [WC] * NCH + [REM]
    starts = [i * WC for i in range(NCH + 1)]
    nchunks = NCH + 1

    def start(ci):
        slot = ci % RING
        w = widths[ci]
        return pltpu.async_copy(
            x_hbm.at[b, :, pl.ds(starts[ci], w)],
            buf.at[slot, :, pl.ds(0, w)],
            sems[slot])

    tcp = pltpu.async_copy(xt_hbm.at[b], tbuf, semt)
    cps = [start(ci) for ci in range(min(RING, nchunks))]

    ms = [jnp.full((L,), -jnp.inf, dtype=jnp.float32) for _ in range(H)]
    aas = [jnp.zeros((L,), dtype=jnp.int32) for _ in range(H)]

    def scan_groups(bufref, gbase, ngroups, ms, aas):
        def body(g, carry):
            mm = list(carry[:H])
            aa = list(carry[H:])
            col = g * L
            gv = jnp.broadcast_to(gbase + g, (L,))
            for r in range(H):
                v = bufref[r, pl.ds(col, L)]
                gt = v > mm[r]
                mm[r] = jnp.where(gt, v, mm[r])
                aa[r] = jnp.where(gt, gv, aa[r])
            return tuple(mm) + tuple(aa)

        carry = lax.fori_loop(0, ngroups, body, tuple(ms) + tuple(aas))
        return list(carry[:H]), list(carry[H:])

    for ci in range(nchunks):
        slot = ci % RING
        cps[slot].wait()
        ms, aas = scan_groups(
            buf.at[slot], starts[ci] // L, widths[ci] // L, ms, aas)
        if ci + RING < nchunks:
            cps[slot] = start(ci + RING)

    tcp.wait()
    ms, aas = scan_groups(tbuf, GTAIL, TB // L, ms, aas)

    resv = jnp.zeros((L,), dtype=jnp.float32)
    for r in range(H):
        rm = ms[r]
        ra = (aas[r] << 4) + iota
        for sh in (8, 4, 2, 1):
            perm = iota ^ sh
            mo = rm.at[perm].get(mode="promise_in_bounds")
            ao = ra.at[perm].get(mode="promise_in_bounds")
            better = (mo > rm) | ((mo == rm) & (ao < ra))
            rm = jnp.where(better, mo, rm)
            ra = jnp.where(better, ao, ra)
        val = ra.astype(jnp.float32) / jnp.float32(N)
        resv = jnp.where(iota == r, val, resv)

    res[...] = resv
    oout = pl.multiple_of(b * L, 8)
    pltpu.sync_copy(res, out_hbm.at[pl.ds(oout, L)])


def kernel(x):
    tail = lax.slice(x, (0, 0, NT * TB), (B, H, N))
    xt = jnp.pad(tail, ((0, 0), (0, 0), (0, TB - TAILC)),
                 constant_values=-jnp.inf)
    out = _argmax_sc(x, xt)
    return out.reshape(B, L)[:, :H]
